# Initial kernel scaffold; baseline (speedup 1.0000x reference)
import jax
import jax.numpy as jnp
from jax.experimental import pallas as pl


def kernel(features, edge_index, W1, b1, W2, b2, W3, b3):
    raise NotImplementedError("write your pallas kernel here")


# speedup vs baseline: 7.6925x; 7.6925x over previous
import jax
import jax.numpy as jnp
from jax import lax
from jax.experimental import pallas as pl
from jax.experimental.pallas import tpu as pltpu
from jax.experimental.pallas import tpu_sc as plsc

N = 10000
E = 320000
D_IN = 128
D_H = 64

NC = 2
NS = 16
NW = NC * NS
CHUNK = 128
CH = -(-E // (NW * CHUNK))
E_PAD = NW * CH * CHUNK
N_PAD = 10240
RPT = N_PAD // NS

_mesh = plsc.VectorSubcoreMesh(core_axis_name="c", subcore_axis_name="s")
_sc_params = pltpu.CompilerParams(use_tc_tiling_on_sc=False)


def _deg_body(src_hbm, dst_hbm, ones_hbm, z1_hbm, out_hbm,
              src_v, dst_v, ones_v, dego_sh, degi_sh):
    c = lax.axis_index("c")
    s = lax.axis_index("s")
    wid = c * NS + s
    pltpu.sync_copy(src_hbm.at[wid], src_v)
    pltpu.sync_copy(dst_hbm.at[wid], dst_v)
    pltpu.sync_copy(ones_hbm, ones_v)
    sl = pl.ds(s * RPT, RPT)
    pltpu.sync_copy(z1_hbm.at[sl], dego_sh.at[sl])
    pltpu.sync_copy(z1_hbm.at[sl], degi_sh.at[sl])
    plsc.subcore_barrier()

    @pl.loop(0, CH)
    def _(j):
        pltpu.sync_copy(ones_v, dego_sh.at[src_v.at[j]], add=True)
        pltpu.sync_copy(ones_v, degi_sh.at[dst_v.at[j]], add=True)

    plsc.subcore_barrier()
    pltpu.sync_copy(dego_sh.at[sl], out_hbm.at[c, 0, sl])
    pltpu.sync_copy(degi_sh.at[sl], out_hbm.at[c, 1, sl])


_deg_call = pl.kernel(
    _deg_body,
    out_type=jax.ShapeDtypeStruct((NC, 2, N_PAD), jnp.float32),
    mesh=_mesh,
    scratch_types=[
        pltpu.VMEM((CH, CHUNK), jnp.int32),
        pltpu.VMEM((CH, CHUNK), jnp.int32),
        pltpu.VMEM((CHUNK,), jnp.float32),
        pltpu.VMEM_SHARED((N_PAD,), jnp.float32),
        pltpu.VMEM_SHARED((N_PAD,), jnp.float32),
    ],
    compiler_params=_sc_params,
)


def _layer_body(h_hbm, src_hbm, dst_hbm, zr_hbm, out_hbm,
                src_v, dst_v, rows_v, agg_sh):
    c = lax.axis_index("c")
    s = lax.axis_index("s")
    wid = c * NS + s
    pltpu.sync_copy(src_hbm.at[wid], src_v)
    pltpu.sync_copy(dst_hbm.at[wid], dst_v)
    sl = pl.ds(s * RPT, RPT)
    pltpu.sync_copy(zr_hbm.at[sl], agg_sh.at[sl])
    plsc.subcore_barrier()

    @pl.loop(0, CH)
    def _(j):
        pltpu.sync_copy(h_hbm.at[src_v.at[j]], rows_v)
        pltpu.sync_copy(rows_v, agg_sh.at[dst_v.at[j]], add=True)

    plsc.subcore_barrier()
    pltpu.sync_copy(agg_sh.at[sl], out_hbm.at[c, sl])


_layer_call = pl.kernel(
    _layer_body,
    out_type=jax.ShapeDtypeStruct((NC, N_PAD, D_H), jnp.float32),
    mesh=_mesh,
    scratch_types=[
        pltpu.VMEM((CH, CHUNK), jnp.int32),
        pltpu.VMEM((CH, CHUNK), jnp.int32),
        pltpu.VMEM((CHUNK, D_H), jnp.float32),
        pltpu.VMEM_SHARED((N_PAD, D_H), jnp.float32),
    ],
    compiler_params=_sc_params,
)


def _prep_body(f_ref, w_ref, degp_ref, xw_ref, ns_ref, nd_ref):
    xw_ref[...] = jnp.dot(f_ref[...], w_ref[...],
                          preferred_element_type=jnp.float32,
                          precision=lax.Precision.HIGHEST)
    dego = degp_ref[0, 0:1, :] + degp_ref[1, 0:1, :]
    degi = degp_ref[0, 1:2, :] + degp_ref[1, 1:2, :]
    ns_ref[...] = jnp.where(dego > 0.0, lax.rsqrt(jnp.maximum(dego, 1.0)), 0.0)
    nd_ref[...] = jnp.where(degi > 0.0, lax.rsqrt(jnp.maximum(degi, 1.0)), 0.0)


_prep_call = pl.pallas_call(
    _prep_body,
    out_shape=(
        jax.ShapeDtypeStruct((N_PAD, D_H), jnp.float32),
        jax.ShapeDtypeStruct((1, N_PAD), jnp.float32),
        jax.ShapeDtypeStruct((1, N_PAD), jnp.float32),
    ),
)


def _scale_body(xw_ref, ns_ref, h_ref):
    h_ref[...] = xw_ref[...] * ns_ref[...]


_scale_call = pl.pallas_call(
    _scale_body,
    out_shape=jax.ShapeDtypeStruct((N_PAD, D_H), jnp.float32),
)


def _combine_body(aggp_ref, nd_ref, ns_ref, b_ref, w_ref, h_ref):
    p = aggp_ref[0] + aggp_ref[1]
    x = jnp.maximum(p * nd_ref[...] + b_ref[...], 0.0)
    h_ref[...] = jnp.dot(x, w_ref[...],
                         preferred_element_type=jnp.float32,
                         precision=lax.Precision.HIGHEST) * ns_ref[...]


_combine_call = pl.pallas_call(
    _combine_body,
    out_shape=jax.ShapeDtypeStruct((N_PAD, D_H), jnp.float32),
)


def _final_body(aggp_ref, nd_ref, b_ref, out_ref):
    p = aggp_ref[0] + aggp_ref[1]
    x = jnp.maximum(p * nd_ref[...] + b_ref[...], 0.0)
    out_ref[...] = jnp.sum(x[:N, :], axis=0, keepdims=True) * (1.0 / N)


_final_call = pl.pallas_call(
    _final_body,
    out_shape=jax.ShapeDtypeStruct((1, D_H), jnp.float32),
)


def kernel(features, edge_index, W1, b1, W2, b2, W3, b3):
    src = edge_index[0]
    dst = edge_index[1]
    padv = jnp.full((E_PAD - E,), N, dtype=jnp.int32)
    srcp = jnp.concatenate([src, padv]).reshape(NW, CH, CHUNK)
    dstp = jnp.concatenate([dst, padv]).reshape(NW, CH, CHUNK)
    featp = jnp.pad(features, ((0, N_PAD - N), (0, 0)))
    ones = jnp.ones((CHUNK,), jnp.float32)
    z1 = jnp.zeros((N_PAD,), jnp.float32)
    zr = jnp.zeros((N_PAD, D_H), jnp.float32)

    degp = _deg_call(srcp, dstp, ones, z1)
    xw1, ns_row, nd_row = _prep_call(featp, W1, degp)
    ns_col = ns_row.reshape(N_PAD, 1)
    nd_col = nd_row.reshape(N_PAD, 1)

    h = _scale_call(xw1, ns_col)
    for bb, Wn in ((b1, W2), (b2, W3)):
        aggp = _layer_call(h, srcp, dstp, zr)
        h = _combine_call(aggp, nd_col, ns_col, bb.reshape(1, D_H), Wn)
    aggp = _layer_call(h, srcp, dstp, zr)
    out = _final_call(aggp, nd_col, b3.reshape(1, D_H))
    return out.reshape(D_H)
